# baseline (speedup 1.0000x reference)
import functools

import jax
import jax.numpy as jnp
from jax import lax
from jax.experimental import pallas as pl
from jax.experimental.pallas import tpu as pltpu
from jax.experimental.pallas import tpu_sc as plsc

D = 128
NC, NS = 2, 16
NW = NC * NS


def _make_gather(batch: int, hist: int):
  bw = batch // NW
  mesh = plsc.VectorSubcoreMesh(core_axis_name="c", subcore_axis_name="s")

  @functools.partial(
      pl.kernel,
      mesh=mesh,
      out_type=jax.ShapeDtypeStruct((hist, batch, D), jnp.float32),
      scratch_types=[
          pltpu.VMEM((hist, bw), jnp.int32),
          pltpu.VMEM((bw, D), jnp.float32),
          pltpu.VMEM((bw, D), jnp.float32),
          pltpu.VMEM((bw, D), jnp.float32),
          pltpu.VMEM((bw, D), jnp.float32),
          pltpu.VMEM((bw, D), jnp.float32),
          pltpu.SemaphoreType.DMA,
          pltpu.SemaphoreType.DMA,
          pltpu.SemaphoreType.DMA,
          pltpu.SemaphoreType.DMA,
          pltpu.SemaphoreType.DMA,
          pltpu.SemaphoreType.DMA,
          pltpu.SemaphoreType.DMA,
          pltpu.SemaphoreType.DMA,
          pltpu.SemaphoreType.DMA,
          pltpu.SemaphoreType.DMA,
      ],
  )
  def gather(xt_hbm, table_hbm, out_hbm, idx_v,
             r0, r1, r2, r3, r4, g0, g1, g2, g3, g4,
             w0, w1, w2, w3, w4):
    wid = lax.axis_index("s") * NC + lax.axis_index("c")
    base = wid * bw
    rows = (r0, r1, r2, r3, r4)
    gsem = (g0, g1, g2, g3, g4)
    wsem = (w0, w1, w2, w3, w4)
    nbuf = 5
    pltpu.sync_copy(xt_hbm.at[:, pl.ds(base, bw)], idx_v)

    def start_g(h, buf):
      pltpu.async_copy(table_hbm.at[idx_v.at[h]], rows[buf], gsem[buf])

    def wait_g(h, buf):
      pltpu.make_async_copy(table_hbm.at[idx_v.at[h]], rows[buf],
                            gsem[buf]).wait()

    def start_w(h, buf):
      pltpu.async_copy(rows[buf], out_hbm.at[h, pl.ds(base, bw)], wsem[buf])

    def wait_w(h, buf):
      pltpu.make_async_copy(rows[buf], out_hbm.at[h, pl.ds(base, bw)],
                            wsem[buf]).wait()

    for h in range(nbuf):
      start_g(h, h)

    def outer(o, carry):
      for buf in range(nbuf):
        h = o * nbuf + buf
        wait_g(h, buf)
        start_w(h, buf)
      for buf in range(nbuf):
        h = o * nbuf + buf
        wait_w(h, buf)

        @pl.when(h + nbuf < hist)
        def _():
          start_g(h + nbuf, buf)
      return carry

    lax.fori_loop(0, hist // nbuf, outer, 0)

  return gather


_gather = _make_gather(4096, 50)


def kernel(x, table):
  xt = jnp.swapaxes(x, 0, 1)
  out_t = _gather(xt, table)
  return jnp.swapaxes(out_t, 0, 1)
